# baseline (speedup 1.0000x reference)
import functools

import jax
import jax.numpy as jnp
from jax import lax
from jax.experimental import pallas as pl
from jax.experimental.pallas import tpu as pltpu
from jax.experimental.pallas import tpu_sc as plsc

F32 = jnp.float32
_PREC = lax.Precision.DEFAULT
_EXACT = lax.Precision.HIGHEST

_NC = 2
_NS = 16
_NW = _NC * _NS
_K = 80


def _splat(v, lane):
    idx = jnp.full((16, 1), lane, jnp.int32)
    dn = lax.GatherDimensionNumbers(
        offset_dims=(), collapsed_slice_dims=(0,), start_index_map=(0,))
    return lax.gather(v, idx, dn, (1,),
                      mode=lax.GatherScatterMode.PROMISE_IN_BOUNDS)


def _make_edge_kernel(n, npad, e, d, nheads):
    ept = e // _NW
    c_chunks = ept // _K
    nj = d // 16
    rpt = npad // _NS

    mesh = plsc.VectorSubcoreMesh(core_axis_name="c", subcore_axis_name="s",
                                  num_cores=_NC, num_subcores=_NS)

    def body(ei_h, tab_h, as_h, ad_h, zd_h, z16_h, outp, denp,
             idxs0, idxd0, ar0, br0, hr0, semg0, semi0,
             idxs1, idxd1, ar1, br1, hr1, semg1, semi1,
             idxs2, idxd2, ar2, br2, hr2, semg2, semi2,
             wb, msg, sidx, acc, dacc, sems):
        cid = lax.axis_index("c")
        sid = lax.axis_index("s")
        wid = sid * _NC + cid

        bufs = ((idxs0, idxd0, ar0, br0, hr0, semg0, semi0),
                (idxs1, idxd1, ar1, br1, hr1, semg1, semi1),
                (idxs2, idxd2, ar2, br2, hr2, semg2, semi2))

        zsl = pl.ds(sid * rpt, rpt)
        pltpu.sync_copy(zd_h, acc.at[zsl])
        pltpu.sync_copy(z16_h, dacc.at[zsl])
        plsc.subcore_barrier()

        def issue_idx(ci, b, sync=False):
            idxs, idxd = b[0], b[1]
            base = wid * ept + ci * _K
            if sync:
                pltpu.sync_copy(ei_h.at[0, pl.ds(base, _K)], idxs)
                pltpu.sync_copy(ei_h.at[1, pl.ds(base, _K)], idxd)
            else:
                pltpu.async_copy(ei_h.at[0, pl.ds(base, _K)], idxs, b[6])
                pltpu.async_copy(ei_h.at[1, pl.ds(base, _K)], idxd, b[6])

        def wait_idx(b):
            pltpu.make_async_copy(ei_h.at[0, pl.ds(0, _K)], b[0], b[6]).wait()
            pltpu.make_async_copy(ei_h.at[1, pl.ds(0, _K)], b[1], b[6]).wait()

        def issue_gathers(b):
            pltpu.async_copy(tab_h.at[b[0]], b[4], b[5])
            pltpu.async_copy(as_h.at[b[0]], b[2], b[5])
            pltpu.async_copy(ad_h.at[b[1]], b[3], b[5])

        def wait_gathers(b):
            pltpu.make_async_copy(tab_h.at[b[0]], b[4], b[5]).wait()
            pltpu.make_async_copy(as_h.at[b[0]], b[2], b[5]).wait()
            pltpu.make_async_copy(ad_h.at[b[1]], b[3], b[5]).wait()

        def wait_scatter():
            pltpu.make_async_copy(wb, dacc.at[sidx], sems).wait()
            pltpu.make_async_copy(msg, acc.at[sidx], sems).wait()

        def step(ci, s):
            b = bufs[s]
            idxd, ar, br, hr = b[1], b[2], b[3], b[4]
            wait_gathers(b)

            @pl.when(ci + 2 < c_chunks)
            def _():
                bn = bufs[(s + 2) % 3]
                wait_idx(bn)
                issue_gathers(bn)

            @pl.when(ci > 0)
            def _():
                wait_scatter()

            for k in range(_K // 16):
                ksl = pl.ds(16 * k, 16)
                sidx[ksl] = idxd[ksl]

            @plsc.parallel_loop(0, _K, 1, unroll=8)
            def _(ei):
                s_ = ar[ei, :] + br[ei, :]
                w = jnp.exp(jnp.maximum(s_, 0.2 * s_))
                wb[ei, :] = w
                for j in range(nj // 2):
                    xi = hr[ei, pl.ds(16 * j, 16)]
                    lo = jnp.bitwise_and(xi, 65535).astype(F32)
                    hi = lax.shift_right_logical(xi, 16).astype(F32)
                    splo = _splat(w, 2 * j if nheads > 1 else 0)
                    sphi = _splat(w, 2 * j + 1 if nheads > 1 else 0)
                    msg[ei, pl.ds(32 * j, 16)] = lo * splo
                    msg[ei, pl.ds(32 * j + 16, 16)] = hi * sphi

            pltpu.async_copy(wb, dacc.at[sidx], sems, add=True)
            pltpu.async_copy(msg, acc.at[sidx], sems, add=True)

            @pl.when(ci + 3 < c_chunks)
            def _():
                issue_idx(ci + 3, b)

        issue_idx(0, bufs[0], sync=True)
        issue_gathers(bufs[0])
        issue_idx(1, bufs[1], sync=True)
        issue_gathers(bufs[1])
        issue_idx(2, bufs[2])

        def triple(j, _):
            step(3 * j, 0)
            step(3 * j + 1, 1)
            step(3 * j + 2, 2)
            return 0

        nt = c_chunks // 3
        lax.fori_loop(0, nt, triple, 0)
        for t in range(c_chunks % 3):
            step(nt * 3 + t, t)

        wait_scatter()
        plsc.subcore_barrier()

        sl = pl.ds(sid * rpt, rpt)
        pltpu.sync_copy(acc.at[sl], outp.at[cid, sl])
        pltpu.sync_copy(dacc.at[sl], denp.at[cid, sl])

    return pl.kernel(
        body,
        out_type=[jax.ShapeDtypeStruct((_NC, npad, d), F32),
                  jax.ShapeDtypeStruct((_NC, npad, 16), F32)],
        mesh=mesh,
        compiler_params=pltpu.CompilerParams(use_tc_tiling_on_sc=False),
        scratch_types=(
            [pltpu.VMEM((_K,), jnp.int32),
             pltpu.VMEM((_K,), jnp.int32),
             pltpu.VMEM((_K, 16), F32),
             pltpu.VMEM((_K, 16), F32),
             pltpu.VMEM((_K, d // 2), jnp.int32),
             pltpu.SemaphoreType.DMA,
             pltpu.SemaphoreType.DMA] * 3 +
            [pltpu.VMEM((_K, 16), F32),
             pltpu.VMEM((_K, d), F32),
             pltpu.VMEM((_K,), jnp.int32),
             pltpu.VMEM_SHARED((npad, d), F32),
             pltpu.VMEM_SHARED((npad, 16), F32),
             pltpu.SemaphoreType.DMA]
        ),
    )


def _tc_a(x_ref, w1_ref, as_ref, ad_ref, h_ref, a1_ref, a2_ref, hm_ref):
    h = jnp.dot(x_ref[...], w1_ref[...], precision=_PREC)
    h_ref[...] = h
    a1_ref[...] = jnp.dot(h, as_ref[...], precision=_EXACT)
    a2_ref[...] = jnp.dot(h, ad_ref[...], precision=_EXACT)
    bm = jnp.max(jnp.abs(h))

    @pl.when(pl.program_id(0) == 0)
    def _():
        hm_ref[0, 0] = bm

    @pl.when(pl.program_id(0) > 0)
    def _():
        hm_ref[0, 0] = jnp.maximum(hm_ref[0, 0], bm)


def _tc_q(h_ref, hm_ref, pe_ref, po_ref, q_ref):
    s = 32767.0 / jnp.maximum(hm_ref[0, 0], 1e-30)
    lo = jnp.dot(h_ref[...], pe_ref[...], precision=_EXACT)
    hi = jnp.dot(h_ref[...], po_ref[...], precision=_EXACT)
    qlo = jnp.round(lo * s).astype(jnp.int32) + 32768
    qhi = jnp.round(hi * s).astype(jnp.int32) + 32768
    q_ref[...] = lax.shift_left(qhi, 16) | qlo


def _tc_b(o_ref, d_ref, r_ref, b1_ref, w2_ref, as_ref, ad_ref, s_ref,
          h2_ref, a1_ref, a2_ref, hm_ref):
    o = o_ref[0] + o_ref[1]
    den = d_ref[0] + d_ref[1]
    denr = jnp.dot(den, r_ref[...], precision=_EXACT)
    sc = jnp.maximum(s_ref[0, 0], 1e-30) / 32767.0
    o = (o - 32768.0 * denr) * sc
    oo = o / (denr + 1e-16) + b1_ref[...]
    hf = jnp.where(oo > 0, oo, jnp.exp(oo) - 1.0)
    h2 = jnp.dot(hf, w2_ref[...], precision=_PREC)
    h2_ref[...] = h2
    a1_ref[...] = jnp.dot(h2, as_ref[...], precision=_EXACT)
    a2_ref[...] = jnp.dot(h2, ad_ref[...], precision=_EXACT)
    bm = jnp.max(jnp.abs(h2))

    @pl.when(pl.program_id(0) == 0)
    def _():
        hm_ref[0, 0] = bm

    @pl.when(pl.program_id(0) > 0)
    def _():
        hm_ref[0, 0] = jnp.maximum(hm_ref[0, 0], bm)


def _tc_c(o_ref, d_ref, r_ref, b2_ref, s_ref, out_ref):
    o = o_ref[0] + o_ref[1]
    den = d_ref[0] + d_ref[1]
    denr = jnp.dot(den, r_ref[...], precision=_EXACT)
    sc = jnp.maximum(s_ref[0, 0], 1e-30) / 32767.0
    o = (o - 32768.0 * denr) * sc
    out_ref[...] = o / (denr + 1e-16) + b2_ref[...]


def kernel(x, edge_index, W1, att_src1, att_dst1, b1, W2, att_src2,
           att_dst2, b2):
    n, d_in = x.shape
    e = edge_index.shape[1]
    d1 = W1.shape[1]
    heads = att_src1.shape[0]
    hid = att_src1.shape[1]
    d2 = W2.shape[1]

    eye = jnp.eye(heads, dtype=F32)
    As1 = jnp.pad((att_src1[:, :, None] * eye[:, None, :]).reshape(d1, heads),
                  ((0, 0), (0, 16 - heads)))
    Ad1 = jnp.pad((att_dst1[:, :, None] * eye[:, None, :]).reshape(d1, heads),
                  ((0, 0), (0, 16 - heads)))
    As2 = jnp.pad(att_src2.T, ((0, 0), (0, 15)))
    Ad2 = jnp.pad(att_dst2.T, ((0, 0), (0, 15)))
    R1 = jnp.pad(jnp.repeat(jnp.eye(heads, dtype=F32), hid, axis=1),
                 ((0, 16 - heads), (0, 0)))
    R2 = jnp.zeros((16, d2), F32).at[0].set(1.0)

    def sel_mats(d):
        k = jnp.arange(d // 2)
        fe = 32 * (k // 16) + k % 16
        rows = jnp.arange(d)[:, None]
        return ((rows == fe[None, :]).astype(F32),
                (rows == (fe + 16)[None, :]).astype(F32))

    Pe1, Po1 = sel_mats(d1)
    Pe2, Po2 = sel_mats(d2)

    npad = ((n + 8 * _NS - 1) // (8 * _NS)) * (8 * _NS)
    rpt = npad // _NS
    zd1 = jnp.zeros((rpt, d1), F32)
    zd2 = jnp.zeros((rpt, d2), F32)
    z16 = jnp.zeros((rpt, 16), F32)

    bn = 1000
    grid = (n // bn,)

    _full = lambda *shape: pl.BlockSpec(shape, lambda i: (0,) * len(shape))
    _smem = lambda: pl.BlockSpec((1, 1), lambda i: (0, 0),
                                 memory_space=pltpu.SMEM)
    _row = lambda *shape: pl.BlockSpec(shape, lambda i: (i, 0))
    _prt = lambda *shape: pl.BlockSpec((_NC,) + shape, lambda i: (0, i, 0))

    h1, as1, ad1, hm1 = pl.pallas_call(
        _tc_a,
        grid=grid,
        in_specs=[_row(bn, d_in), _full(d_in, d1),
                  _full(d1, 16), _full(d1, 16)],
        out_specs=[_row(bn, d1), _row(bn, 16), _row(bn, 16), _smem()],
        out_shape=[jax.ShapeDtypeStruct((n, d1), F32),
                   jax.ShapeDtypeStruct((n, 16), F32),
                   jax.ShapeDtypeStruct((n, 16), F32),
                   jax.ShapeDtypeStruct((1, 1), F32)],
    )(x, W1, As1, Ad1)

    q1 = pl.pallas_call(
        _tc_q,
        grid=grid,
        in_specs=[_row(bn, d1), _smem(),
                  _full(d1, d1 // 2), _full(d1, d1 // 2)],
        out_specs=_row(bn, d1 // 2),
        out_shape=jax.ShapeDtypeStruct((n, d1 // 2), jnp.int32),
    )(h1, hm1, Pe1, Po1)

    o1p, d1p = _make_edge_kernel(n, npad, e, d1, heads)(
        edge_index, q1, as1, ad1, zd1, z16)

    h2, as2, ad2, hm2 = pl.pallas_call(
        _tc_b,
        grid=grid,
        in_specs=[_prt(bn, d1), _prt(bn, 16), _full(16, d1), _full(1, d1),
                  _full(d1, d2), _full(d2, 16), _full(d2, 16), _smem()],
        out_specs=[_row(bn, d2), _row(bn, 16), _row(bn, 16), _smem()],
        out_shape=[jax.ShapeDtypeStruct((n, d2), F32),
                   jax.ShapeDtypeStruct((n, 16), F32),
                   jax.ShapeDtypeStruct((n, 16), F32),
                   jax.ShapeDtypeStruct((1, 1), F32)],
    )(o1p, d1p, R1, b1.reshape(1, d1), W2, As2, Ad2, hm1)

    q2 = pl.pallas_call(
        _tc_q,
        grid=grid,
        in_specs=[_row(bn, d2), _smem(),
                  _full(d2, d2 // 2), _full(d2, d2 // 2)],
        out_specs=_row(bn, d2 // 2),
        out_shape=jax.ShapeDtypeStruct((n, d2 // 2), jnp.int32),
    )(h2, hm2, Pe2, Po2)

    o2p, d2p = _make_edge_kernel(n, npad, e, d2, 1)(
        edge_index, q2, as2, ad2, zd2, z16)

    out = pl.pallas_call(
        _tc_c,
        grid=grid,
        in_specs=[_prt(bn, d2), _prt(bn, 16), _full(16, d2), _full(1, d2),
                  _smem()],
        out_specs=_row(bn, d2),
        out_shape=jax.ShapeDtypeStruct((n, d2), F32),
    )(o2p, d2p, R2, b2.reshape(1, d2), hm2)

    return out


# speedup vs baseline: 1.0320x; 1.0320x over previous
import functools

import jax
import jax.numpy as jnp
from jax import lax
from jax.experimental import pallas as pl
from jax.experimental.pallas import tpu as pltpu
from jax.experimental.pallas import tpu_sc as plsc

F32 = jnp.float32
_PREC = lax.Precision.DEFAULT
_EXACT = lax.Precision.HIGHEST

_NC = 2
_NS = 16
_NW = _NC * _NS
_K = 80


def _splat(v, lane):
    idx = jnp.full((16, 1), lane, jnp.int32)
    dn = lax.GatherDimensionNumbers(
        offset_dims=(), collapsed_slice_dims=(0,), start_index_map=(0,))
    return lax.gather(v, idx, dn, (1,),
                      mode=lax.GatherScatterMode.PROMISE_IN_BOUNDS)


def _make_edge_kernel(n, npad, e, d, nheads):
    ept = e // _NW
    c_chunks = ept // _K
    nj = d // 16
    rpt = npad // _NS

    mesh = plsc.VectorSubcoreMesh(core_axis_name="c", subcore_axis_name="s",
                                  num_cores=_NC, num_subcores=_NS)

    def body(ei_h, tab_h, as_h, ad_h, zd_h, outp,
             ib0, ar0, br0, hr0, semg0, semi0,
             ib1, ar1, br1, hr1, semg1, semi1,
             ib2, ar2, br2, hr2, semg2, semi2,
             msg, sidx, acc, sems):
        cid = lax.axis_index("c")
        sid = lax.axis_index("s")
        wid = sid * _NC + cid

        bufs = ((ib0, ar0, br0, hr0, semg0, semi0),
                (ib1, ar1, br1, hr1, semg1, semi1),
                (ib2, ar2, br2, hr2, semg2, semi2))

        zsl = pl.ds(sid * rpt, rpt)
        pltpu.sync_copy(zd_h, acc.at[zsl])
        plsc.subcore_barrier()

        def issue_idx(ci, b, sync=False):
            base = wid * ept + ci * _K
            if sync:
                pltpu.sync_copy(ei_h.at[:, pl.ds(base, _K)], b[0])
            else:
                pltpu.async_copy(ei_h.at[:, pl.ds(base, _K)], b[0], b[5])

        def wait_idx(b):
            pltpu.make_async_copy(ei_h.at[:, pl.ds(0, _K)], b[0], b[5]).wait()

        def issue_gathers(b):
            pltpu.async_copy(tab_h.at[b[0].at[0]], b[3], b[4])
            pltpu.async_copy(as_h.at[b[0].at[0]], b[1], b[4])
            pltpu.async_copy(ad_h.at[b[0].at[1]], b[2], b[4])

        def wait_gathers(b):
            pltpu.make_async_copy(tab_h.at[b[0].at[0]], b[3], b[4]).wait()
            pltpu.make_async_copy(as_h.at[b[0].at[0]], b[1], b[4]).wait()
            pltpu.make_async_copy(ad_h.at[b[0].at[1]], b[2], b[4]).wait()

        def wait_scatter():
            pltpu.make_async_copy(msg, acc.at[sidx], sems).wait()

        def step(ci, s):
            b = bufs[s]
            ib, ar, br, hr = b[0], b[1], b[2], b[3]
            wait_gathers(b)

            @pl.when(ci + 2 < c_chunks)
            def _():
                bn = bufs[(s + 2) % 3]
                wait_idx(bn)
                issue_gathers(bn)

            @pl.when(ci > 0)
            def _():
                wait_scatter()

            for k in range(_K // 16):
                ksl = pl.ds(16 * k, 16)
                sidx[ksl] = ib[1, ksl]

            @plsc.parallel_loop(0, _K, 1, unroll=8)
            def _(ei):
                s_ = ar[ei, :] + br[ei, :]
                w = jnp.exp(jnp.maximum(s_, 0.2 * s_))
                msg[ei, pl.ds(d, 16)] = w
                for j in range(nj // 2):
                    xi = hr[ei, pl.ds(16 * j, 16)]
                    lo = jnp.bitwise_and(xi, 65535).astype(F32)
                    hi = lax.shift_right_logical(xi, 16).astype(F32)
                    splo = _splat(w, j if nheads > 1 else 0)
                    sphi = _splat(w, nj // 2 + j if nheads > 1 else 0)
                    msg[ei, pl.ds(16 * j, 16)] = lo * splo
                    msg[ei, pl.ds(d // 2 + 16 * j, 16)] = hi * sphi

            pltpu.async_copy(msg, acc.at[sidx], sems, add=True)

            @pl.when(ci + 3 < c_chunks)
            def _():
                issue_idx(ci + 3, b)

        issue_idx(0, bufs[0], sync=True)
        issue_gathers(bufs[0])
        issue_idx(1, bufs[1], sync=True)
        issue_gathers(bufs[1])
        issue_idx(2, bufs[2])

        def triple(j, _):
            step(3 * j, 0)
            step(3 * j + 1, 1)
            step(3 * j + 2, 2)
            return 0

        nt = c_chunks // 3
        lax.fori_loop(0, nt, triple, 0)
        for t in range(c_chunks % 3):
            step(nt * 3 + t, t)

        wait_scatter()
        plsc.subcore_barrier()

        sl = pl.ds(sid * rpt, rpt)
        pltpu.sync_copy(acc.at[sl], outp.at[cid, sl])

    return pl.kernel(
        body,
        out_type=jax.ShapeDtypeStruct((_NC, npad, d + 16), F32),
        mesh=mesh,
        compiler_params=pltpu.CompilerParams(use_tc_tiling_on_sc=False),
        scratch_types=(
            [pltpu.VMEM((2, _K), jnp.int32),
             pltpu.VMEM((_K, 16), F32),
             pltpu.VMEM((_K, 16), F32),
             pltpu.VMEM((_K, d // 2), jnp.int32),
             pltpu.SemaphoreType.DMA,
             pltpu.SemaphoreType.DMA] * 3 +
            [pltpu.VMEM((_K, d + 16), F32),
             pltpu.VMEM((_K,), jnp.int32),
             pltpu.VMEM_SHARED((npad, d + 16), F32),
             pltpu.SemaphoreType.DMA]
        ),
    )


def _tc_a(x_ref, w1_ref, as_ref, ad_ref, h_ref, a1_ref, a2_ref, hm_ref):
    h = jnp.dot(x_ref[...], w1_ref[...], precision=_PREC)
    h_ref[...] = h
    a1_ref[...] = jnp.dot(h, as_ref[...], precision=_EXACT)
    a2_ref[...] = jnp.dot(h, ad_ref[...], precision=_EXACT)
    bm = jnp.max(jnp.abs(h))

    @pl.when(pl.program_id(0) == 0)
    def _():
        hm_ref[0, 0] = bm

    @pl.when(pl.program_id(0) > 0)
    def _():
        hm_ref[0, 0] = jnp.maximum(hm_ref[0, 0], bm)


def _tc_q(h_ref, hm_ref, q_ref):
    s = 32767.0 / jnp.maximum(hm_ref[0, 0], 1e-30)
    d = h_ref.shape[1]
    lo = h_ref[:, : d // 2]
    hi = h_ref[:, d // 2:]
    qlo = jnp.round(lo * s).astype(jnp.int32) + 32768
    qhi = jnp.round(hi * s).astype(jnp.int32) + 32768
    q_ref[...] = lax.shift_left(qhi, 16) | qlo


def _tc_b(p_ref, r_ref, b1_ref, w2_ref, as_ref, ad_ref, s_ref,
          h2_ref, a1_ref, a2_ref, hm_ref):
    d = p_ref.shape[2] - 16
    full = p_ref[0] + p_ref[1]
    o = full[:, :d]
    den = full[:, d:]
    denr = jnp.dot(den, r_ref[...], precision=_EXACT)
    sc = jnp.maximum(s_ref[0, 0], 1e-30) / 32767.0
    o = (o - 32768.0 * denr) * sc
    oo = o / (denr + 1e-16) + b1_ref[...]
    hf = jnp.where(oo > 0, oo, jnp.exp(oo) - 1.0)
    h2 = jnp.dot(hf, w2_ref[...], precision=_PREC)
    h2_ref[...] = h2
    a1_ref[...] = jnp.dot(h2, as_ref[...], precision=_EXACT)
    a2_ref[...] = jnp.dot(h2, ad_ref[...], precision=_EXACT)
    bm = jnp.max(jnp.abs(h2))

    @pl.when(pl.program_id(0) == 0)
    def _():
        hm_ref[0, 0] = bm

    @pl.when(pl.program_id(0) > 0)
    def _():
        hm_ref[0, 0] = jnp.maximum(hm_ref[0, 0], bm)


def _tc_c(p_ref, r_ref, b2_ref, s_ref, out_ref):
    d = p_ref.shape[2] - 16
    full = p_ref[0] + p_ref[1]
    o = full[:, :d]
    den = full[:, d:]
    denr = jnp.dot(den, r_ref[...], precision=_EXACT)
    sc = jnp.maximum(s_ref[0, 0], 1e-30) / 32767.0
    o = (o - 32768.0 * denr) * sc
    out_ref[...] = o / (denr + 1e-16) + b2_ref[...]


def kernel(x, edge_index, W1, att_src1, att_dst1, b1, W2, att_src2,
           att_dst2, b2):
    n, d_in = x.shape
    e = edge_index.shape[1]
    d1 = W1.shape[1]
    heads = att_src1.shape[0]
    hid = att_src1.shape[1]
    d2 = W2.shape[1]

    eye = jnp.eye(heads, dtype=F32)
    As1 = jnp.pad((att_src1[:, :, None] * eye[:, None, :]).reshape(d1, heads),
                  ((0, 0), (0, 16 - heads)))
    Ad1 = jnp.pad((att_dst1[:, :, None] * eye[:, None, :]).reshape(d1, heads),
                  ((0, 0), (0, 16 - heads)))
    As2 = jnp.pad(att_src2.T, ((0, 0), (0, 15)))
    Ad2 = jnp.pad(att_dst2.T, ((0, 0), (0, 15)))
    R1 = jnp.pad(jnp.repeat(jnp.eye(heads, dtype=F32), hid, axis=1),
                 ((0, 16 - heads), (0, 0)))
    R2 = jnp.zeros((16, d2), F32).at[0].set(1.0)

    npad = ((n + 8 * _NS - 1) // (8 * _NS)) * (8 * _NS)
    rpt = npad // _NS
    zd1 = jnp.zeros((rpt, d1 + 16), F32)
    zd2 = jnp.zeros((rpt, d2 + 16), F32)

    bn = 1000
    grid = (n // bn,)

    _full = lambda *shape: pl.BlockSpec(shape, lambda i: (0,) * len(shape))
    _smem = lambda: pl.BlockSpec((1, 1), lambda i: (0, 0),
                                 memory_space=pltpu.SMEM)
    _row = lambda *shape: pl.BlockSpec(shape, lambda i: (i, 0))
    _prt = lambda *shape: pl.BlockSpec((_NC,) + shape, lambda i: (0, i, 0))

    h1, as1, ad1, hm1 = pl.pallas_call(
        _tc_a,
        grid=grid,
        in_specs=[_row(bn, d_in), _full(d_in, d1),
                  _full(d1, 16), _full(d1, 16)],
        out_specs=[_row(bn, d1), _row(bn, 16), _row(bn, 16), _smem()],
        out_shape=[jax.ShapeDtypeStruct((n, d1), F32),
                   jax.ShapeDtypeStruct((n, 16), F32),
                   jax.ShapeDtypeStruct((n, 16), F32),
                   jax.ShapeDtypeStruct((1, 1), F32)],
    )(x, W1, As1, Ad1)

    q1 = pl.pallas_call(
        _tc_q,
        grid=grid,
        in_specs=[_row(bn, d1), _smem()],
        out_specs=_row(bn, d1 // 2),
        out_shape=jax.ShapeDtypeStruct((n, d1 // 2), jnp.int32),
    )(h1, hm1)

    o1p = _make_edge_kernel(n, npad, e, d1, heads)(
        edge_index, q1, as1, ad1, zd1)

    h2, as2, ad2, hm2 = pl.pallas_call(
        _tc_b,
        grid=grid,
        in_specs=[_prt(bn, d1 + 16), _full(16, d1), _full(1, d1),
                  _full(d1, d2), _full(d2, 16), _full(d2, 16), _smem()],
        out_specs=[_row(bn, d2), _row(bn, 16), _row(bn, 16), _smem()],
        out_shape=[jax.ShapeDtypeStruct((n, d2), F32),
                   jax.ShapeDtypeStruct((n, 16), F32),
                   jax.ShapeDtypeStruct((n, 16), F32),
                   jax.ShapeDtypeStruct((1, 1), F32)],
    )(o1p, R1, b1.reshape(1, d1), W2, As2, Ad2, hm1)

    q2 = pl.pallas_call(
        _tc_q,
        grid=grid,
        in_specs=[_row(bn, d2), _smem()],
        out_specs=_row(bn, d2 // 2),
        out_shape=jax.ShapeDtypeStruct((n, d2 // 2), jnp.int32),
    )(h2, hm2)

    o2p = _make_edge_kernel(n, npad, e, d2, 1)(
        edge_index, q2, as2, ad2, zd2)

    out = pl.pallas_call(
        _tc_c,
        grid=grid,
        in_specs=[_prt(bn, d2 + 16), _full(16, d2), _full(1, d2),
                  _smem()],
        out_specs=_row(bn, d2),
        out_shape=jax.ShapeDtypeStruct((n, d2), F32),
    )(o2p, R2, b2.reshape(1, d2), hm2)

    return out
